# Initial kernel scaffold; baseline (speedup 1.0000x reference)
import jax
import jax.numpy as jnp
from jax.experimental import pallas as pl


def kernel(li_bev_coors, ra_bev_coors, ra_points, ra_voxel_coords):
    raise NotImplementedError("write your pallas kernel here")


# speedup vs baseline: 2.7346x; 2.7346x over previous
import functools

import jax
import jax.numpy as jnp
from jax import lax
from jax.experimental import pallas as pl
from jax.experimental.pallas import tpu as pltpu
from jax.experimental.pallas import tpu_sc as plsc

_NC = 2
_NS = 16
_NW = _NC * _NS
_L = 16

_GRID = 513
_SENTINEL = 30000.0

_SX = (0, -1, 1)
_SY = (0, 1, -1)


def _wrap(v):
    v = jnp.where(v < 0, v + _GRID, v)
    return jnp.where(v >= _GRID, v - _GRID, v)


def _vfull(v, dtype=jnp.float32):
    return jnp.full((_L,), v, dtype)


def _newton_sqrt(x):
    i = lax.bitcast_convert_type(x, jnp.int32)
    y = lax.bitcast_convert_type(
        lax.shift_right_logical(i, _vfull(1, jnp.int32))
        + _vfull(0x1FBD1DF5, jnp.int32),
        jnp.float32,
    )
    half = _vfull(0.5)
    y = half * (y + x / y)
    y = half * (y + x / y)
    return y


def _gauss_sc(lix, liy, rax, ray, vx, vy, p4):
    n_li = lix.shape[0]
    n_ra = rax.shape[0]
    n_pt = vx.shape[0]
    nb_li = n_li // _NW
    nb_ra = n_ra // _NW
    mesh = plsc.VectorSubcoreMesh(core_axis_name="c", subcore_axis_name="s")

    @functools.partial(
        pl.kernel,
        mesh=mesh,
        out_type=[
            jax.ShapeDtypeStruct((n_li * _L,), jnp.float32),
            jax.ShapeDtypeStruct((n_ra * _L,), jnp.float32),
        ],
        scratch_types=[
            pltpu.VMEM((n_pt,), jnp.float32),
            pltpu.VMEM((n_pt,), jnp.float32),
            pltpu.VMEM((n_pt,), jnp.float32),
            pltpu.VMEM((n_pt,), jnp.float32),
            pltpu.SMEM((nb_li,), jnp.int32),
            pltpu.SMEM((nb_li,), jnp.int32),
            pltpu.SMEM((nb_ra,), jnp.int32),
            pltpu.SMEM((nb_ra,), jnp.int32),
            pltpu.VMEM((nb_li * _L,), jnp.float32),
            pltpu.VMEM((nb_ra * _L,), jnp.float32),
            pltpu.VMEM((nb_li,), jnp.int32),
        ],
    )
    def k(
        lix_hbm,
        liy_hbm,
        rax_hbm,
        ray_hbm,
        vx_hbm,
        vy_hbm,
        p4_hbm,
        li_out_hbm,
        ra_out_hbm,
        px_v,
        py_v,
        pn_v,
        p4_v,
        bxl_v,
        byl_v,
        bxr_v,
        byr_v,
        ol_v,
        or_v,
        tmp_v,
    ):
        wid = lax.axis_index("s") * _NC + lax.axis_index("c")

        pltpu.sync_copy(vx_hbm, px_v)
        pltpu.sync_copy(vy_hbm, py_v)
        pltpu.sync_copy(p4_hbm, p4_v)
        for hbm, nb_c, smem in (
            (lix_hbm, nb_li, bxl_v),
            (liy_hbm, nb_li, byl_v),
            (rax_hbm, nb_ra, bxr_v),
            (ray_hbm, nb_ra, byr_v),
        ):
            pltpu.sync_copy(
                hbm.at[pl.ds(wid * nb_c, nb_c)], tmp_v.at[pl.ds(0, nb_c)]
            )
            for g in range(nb_c // _L):
                vec = tmp_v[pl.ds(g * _L, _L)]
                for t in range(_L):
                    smem[g * _L + t] = vec[t]

        sent_v = _vfull(_SENTINEL)
        thresh_v = _vfull(0.1)
        ones_i = _vfull(1, jnp.int32)
        zero_i = _vfull(0, jnp.int32)
        lane = lax.iota(jnp.int32, _L)
        bfly = [lane ^ _vfull(k, jnp.int32) for k in (1, 2, 4, 8)]

        @plsc.parallel_loop(0, n_pt, _L, carry=zero_i)
        def _prep(i, cnt):
            sl = pl.ds(i, _L)
            valid = jnp.abs(p4_v[sl]) > thresh_v
            px = jnp.where(valid, px_v[sl], sent_v)
            py = jnp.where(valid, py_v[sl], sent_v)
            px_v[sl] = px
            py_v[sl] = py
            pn_v[sl] = px * px + py * py
            return cnt + jnp.where(valid, ones_i, zero_i)

        cnt = _prep
        for p in bfly:
            cnt = cnt + jnp.take(cnt, p)
        scale_v = jnp.where(cnt > ones_i, _vfull(0.01), _vfull(0.0))

        lane_is = [lane == _vfull(j, jnp.int32) for j in range(9)]

        def do_bases(nb, bx_v, by_v, out_v):
            def base_body(b, _):
                bx = bx_v[b]
                by = by_v[b]
                qx = [_wrap(bx + s).astype(jnp.float32) for s in _SX]
                qy = [_wrap(by + s).astype(jnp.float32) for s in _SY]
                m2x = [
                    jnp.broadcast_to(jnp.float32(-2.0) * q, (_L,)) for q in qx
                ]
                m2y = [
                    jnp.broadcast_to(jnp.float32(-2.0) * q, (_L,)) for q in qy
                ]
                init = tuple(_vfull(3e38) for _ in range(9))

                @plsc.parallel_loop(0, n_pt, _L, unroll=2, carry=init)
                def accs(i, acc):
                    sl = pl.ds(i, _L)
                    pxc = px_v[sl]
                    pyc = py_v[sl]
                    pnc = pn_v[sl]
                    u = [pnc + m2x[kk] * pxc for kk in range(3)]
                    w = [m2y[ll] * pyc for ll in range(3)]
                    return tuple(
                        jnp.minimum(acc[3 * ll + kk], u[kk] + w[ll])
                        for ll in range(3)
                        for kk in range(3)
                    )

                r = jnp.zeros((_L,), jnp.float32)
                for ll in range(3):
                    for kk in range(3):
                        j = 3 * ll + kk
                        qn = qx[kk] * qx[kk] + qy[ll] * qy[ll]
                        qn_v = jnp.broadcast_to(qn, (_L,))
                        m = accs[j]
                        for p in bfly:
                            m = jnp.minimum(m, jnp.take(m, p))
                        r = jnp.where(lane_is[j], m + qn_v, r)
                out_v[pl.ds(b * _L, _L)] = r
                return 0

            lax.fori_loop(0, nb, base_body, 0)

            @plsc.parallel_loop(0, nb * _L, _L)
            def _fin(i):
                sl = pl.ds(i, _L)
                out_v[sl] = _newton_sqrt(out_v[sl]) * scale_v

        do_bases(nb_li, bxl_v, byl_v, ol_v)
        do_bases(nb_ra, bxr_v, byr_v, or_v)

        pltpu.sync_copy(
            ol_v, li_out_hbm.at[pl.ds(wid * nb_li * _L, nb_li * _L)]
        )
        pltpu.sync_copy(
            or_v, ra_out_hbm.at[pl.ds(wid * nb_ra * _L, nb_ra * _L)]
        )

    return k(lix, liy, rax, ray, vx, vy, p4)


def kernel(li_bev_coors, ra_bev_coors, ra_points, ra_voxel_coords):
    lidar_out = []
    radar_out = []
    B = ra_points.shape[0]
    for b in range(B):
        li = li_bev_coors[b].astype(jnp.int32)
        ra = ra_bev_coors[b].astype(jnp.int32)
        p4 = ra_points[b, :, 4].astype(jnp.float32)
        vx = ra_voxel_coords[b, :, 1].astype(jnp.float32)
        vy = ra_voxel_coords[b, :, 2].astype(jnp.float32)
        li_flat, ra_flat = _gauss_sc(
            li[:, 0], li[:, 1], ra[:, 0], ra[:, 1], vx, vy, p4
        )
        lidar_out.append(li_flat.reshape(li.shape[0], 16)[:, :9])
        radar_out.append(ra_flat.reshape(ra.shape[0], 16)[:, :9])
    return (tuple(lidar_out), tuple(radar_out))
